# baseline (speedup 1.0000x reference)
import functools

import jax
import jax.numpy as jnp
from jax import lax
from jax.experimental import pallas as pl
from jax.experimental.pallas import tpu as pltpu
from jax.experimental.pallas import tpu_sc as plsc

N = 10000
E = 320000
D = 128
DOUT = 12

NC = 2
NS = 16
NW = NC * NS
K = 128
NBUF = 2
CHUNKS = ((-(-E // (NW * K)) + NBUF - 1) // NBUF) * NBUF
EPAD = CHUNKS * NW * K
NPAD = (N // 128 + 1) * 128
RPT = NPAD // NS

_mesh = plsc.VectorSubcoreMesh(core_axis_name="c", subcore_axis_name="s")

def _fill_2d(ref, rows, val):
    val16 = jnp.full((16,), val, jnp.float32)
    def body(i, _):
        r = i // (D // 16)
        col = (i % (D // 16)) * 16
        ref[r, pl.ds(col, 16)] = val16
        return 0
    lax.fori_loop(0, rows * (D // 16), body, 0)


def _fill_1d(ref, n, val):
    val16 = jnp.full((16,), val, jnp.float32)
    def body(i, _):
        ref[pl.ds(i * 16, 16)] = val16
        return 0
    lax.fori_loop(0, n // 16, body, 0)


@functools.partial(
    pl.kernel,
    out_type=jax.ShapeDtypeStruct((NC * NPAD,), jnp.float32),
    mesh=_mesh,
    scratch_types=[
        pltpu.VMEM((CHUNKS, K), jnp.int32),
        pltpu.VMEM((K,), jnp.float32),
        pltpu.VMEM((((RPT + 15) // 16) * 16,), jnp.float32),
        pltpu.VMEM_SHARED((NPAD,), jnp.float32),
        pltpu.SemaphoreType.DMA,
        pltpu.SemaphoreType.DMA,
    ],
)
def _sc_degree(dst_hbm, out_hbm, didx_v, ones_v, zrow_v, acc, semi, sems):
    c = lax.axis_index("c")
    s = lax.axis_index("s")
    wid = s * NC + c

    pltpu.async_copy(dst_hbm.at[pl.ds(wid * CHUNKS, CHUNKS)], didx_v, semi)
    _fill_1d(ones_v, K, 1.0)
    _fill_1d(zrow_v, ((RPT + 15) // 16) * 16, 0.0)
    pltpu.sync_copy(zrow_v.at[pl.ds(0, RPT)], acc.at[pl.ds(s * RPT, RPT)])
    pltpu.make_async_copy(dst_hbm.at[pl.ds(0, CHUNKS)], didx_v, semi).wait()
    plsc.subcore_barrier()

    def body(j, _):
        pltpu.async_copy(ones_v, acc.at[didx_v.at[j]], sems, add=True)
        return 0

    lax.fori_loop(0, CHUNKS, body, 0)

    def drain(j, _):
        pltpu.make_async_copy(ones_v, acc.at[didx_v.at[0]], sems).wait()
        return 0

    lax.fori_loop(0, CHUNKS, drain, 0)
    plsc.subcore_barrier()
    pltpu.sync_copy(acc.at[pl.ds(s * RPT, RPT)], zrow_v.at[pl.ds(0, RPT)])
    pltpu.sync_copy(zrow_v.at[pl.ds(0, RPT)],
                    out_hbm.at[pl.ds(c * NPAD + s * RPT, RPT)])


@functools.partial(
    pl.kernel,
    out_type=jax.ShapeDtypeStruct((NC, NPAD, D), jnp.float32),
    mesh=_mesh,
    scratch_types=[
        pltpu.VMEM((2, NBUF, K), jnp.int32),
        pltpu.VMEM((2, NBUF, K), jnp.int32),
        pltpu.VMEM((NBUF, K, D), jnp.float32),
        pltpu.VMEM_SHARED((NPAD, D), jnp.float32),
        pltpu.SemaphoreType.DMA,
        [pltpu.SemaphoreType.DMA] * NBUF,
        [pltpu.SemaphoreType.DMA] * NBUF,
    ],
)
def _sc_segsum(y_hbm, src_hbm, dst_hbm, out_hbm, sidx_v, didx_v, rows_v, acc,
               semi, semg, sems):
    c = lax.axis_index("c")
    s = lax.axis_index("s")
    wid = s * NC + c
    base = wid * CHUNKS
    G = CHUNKS // NBUF

    def istart(g, p):
        pltpu.async_copy(src_hbm.at[pl.ds(base + g * NBUF, NBUF)],
                         sidx_v.at[p], semi)
        pltpu.async_copy(dst_hbm.at[pl.ds(base + g * NBUF, NBUF)],
                         didx_v.at[p], semi)

    def iwait():
        pltpu.make_async_copy(src_hbm.at[pl.ds(0, NBUF)], sidx_v.at[0],
                              semi).wait()
        pltpu.make_async_copy(dst_hbm.at[pl.ds(0, NBUF)], didx_v.at[0],
                              semi).wait()

    istart(0, 0)

    val16 = jnp.zeros((16,), jnp.float32)

    def zbody(i, _):
        r = i // (D // 16)
        col = (i % (D // 16)) * 16
        rows_v[0, r, pl.ds(col, 16)] = val16
        return 0

    lax.fori_loop(0, K * (D // 16), zbody, 0)
    rbase = s * RPT
    for j in range(RPT // K):
        pltpu.sync_copy(rows_v.at[0], acc.at[pl.ds(rbase + j * K, K)])
    rem = RPT % K
    if rem:
        pltpu.sync_copy(rows_v.at[0, pl.ds(0, rem)],
                        acc.at[pl.ds(rbase + (RPT // K) * K, rem)])
    iwait()
    plsc.subcore_barrier()

    def gstart(p, b):
        pltpu.async_copy(y_hbm.at[sidx_v.at[p, b]], rows_v.at[b], semg[b])

    def gwait(b):
        pltpu.make_async_copy(y_hbm.at[sidx_v.at[0, 0]], rows_v.at[b],
                              semg[b]).wait()

    def sstart(p, b):
        pltpu.async_copy(rows_v.at[b], acc.at[didx_v.at[p, b]], sems[b],
                         add=True)

    def swait(b):
        pltpu.make_async_copy(rows_v.at[b], acc.at[didx_v.at[0, 0]],
                              sems[b]).wait()

    for b in range(NBUF):
        gstart(0, b)

    def outer(g, _):
        p = lax.rem(g, 2)
        pn = 1 - p
        more = g + 1 < G

        @pl.when(more)
        def _():
            istart(g + 1, pn)

        for b in range(NBUF):
            gwait(b)
            sstart(p, b)

        @pl.when(more)
        def _():
            iwait()

        for b in range(NBUF):
            swait(b)

            @pl.when(more)
            def _():
                gstart(pn, b)
        return 0

    lax.fori_loop(0, G, outer, 0)
    plsc.subcore_barrier()
    pltpu.sync_copy(acc.at[pl.ds(rbase, RPT)], out_hbm.at[c, pl.ds(rbase, RPT)])


_R = 1000


def _tc_dense1_body(x_ref, w_ref, c0_ref, c1_ref, y_ref, dinv_ref):
    deg = c0_ref[...] + c1_ref[...] + 1.0
    dinv = lax.rsqrt(deg)
    dinv_ref[...] = dinv
    y_ref[...] = jnp.dot(x_ref[...], w_ref[...],
                         preferred_element_type=jnp.float32) * dinv


def _tc_dense_mid_body(z0_ref, z1_ref, y_ref, dinv_ref, b_ref, w_ref, out_ref):
    dinv = dinv_ref[...]
    h = dinv * (z0_ref[...] + z1_ref[...] + y_ref[...]) + b_ref[...]
    h = jnp.maximum(h, 0.0)
    out_ref[...] = jnp.dot(h, w_ref[...],
                           preferred_element_type=jnp.float32) * dinv


def _tc_dense_out_body(z0_ref, z1_ref, y_ref, dinv_ref, b_ref, m_ref,
                       wr_ref, br_ref, out_ref):
    h = dinv_ref[...] * (z0_ref[...] + z1_ref[...] + y_ref[...]) + b_ref[...]
    h = jnp.maximum(h, 0.0) * m_ref[...]
    out_ref[...] = jnp.dot(h, wr_ref[...],
                           preferred_element_type=jnp.float32) + br_ref[...]


def _rows_spec(width):
    return pl.BlockSpec((_R, width), lambda i: (i, 0))


def _full_spec(shape):
    return pl.BlockSpec(shape, lambda i: (0,) * len(shape))


def kernel(x, edge_index, target_mask, W1, b1, W2, b2, Wr, br):
    src = edge_index[0]
    dst = edge_index[1]
    pad = EPAD - E
    src_p = jnp.concatenate([src, jnp.zeros((pad,), jnp.int32)])
    src_p = src_p.reshape(NW * CHUNKS, K)
    dst_p = jnp.concatenate([dst, jnp.full((pad,), N, jnp.int32)])
    dst_p = dst_p.reshape(NW * CHUNKS, K)

    cnt = _sc_degree(dst_p)
    c0 = cnt[:N, None]
    c1 = cnt[NPAD:NPAD + N, None]

    grid = N // _R
    y1, dinv = pl.pallas_call(
        _tc_dense1_body,
        grid=(grid,),
        in_specs=[_rows_spec(D), _full_spec((D, D)), _rows_spec(1), _rows_spec(1)],
        out_specs=[_rows_spec(D), _rows_spec(1)],
        out_shape=[jax.ShapeDtypeStruct((N, D), jnp.float32),
                   jax.ShapeDtypeStruct((N, 1), jnp.float32)],
    )(x, W1, c0, c1)

    z = _sc_segsum(y1, src_p, dst_p)

    y2 = pl.pallas_call(
        _tc_dense_mid_body,
        grid=(grid,),
        in_specs=[_rows_spec(D), _rows_spec(D), _rows_spec(D), _rows_spec(1),
                  _full_spec((1, D)), _full_spec((D, D))],
        out_specs=_rows_spec(D),
        out_shape=jax.ShapeDtypeStruct((N, D), jnp.float32),
    )(z[0, :N], z[1, :N], y1, dinv, b1[None, :], W2)

    z2 = _sc_segsum(y2, src_p, dst_p)

    mask_f = target_mask.astype(jnp.float32)[:, None]
    pred = pl.pallas_call(
        _tc_dense_out_body,
        grid=(grid,),
        in_specs=[_rows_spec(D), _rows_spec(D), _rows_spec(D), _rows_spec(1),
                  _full_spec((1, D)), _rows_spec(1),
                  _full_spec((D, DOUT)), _full_spec((1, DOUT))],
        out_specs=_rows_spec(DOUT),
        out_shape=jax.ShapeDtypeStruct((N, DOUT), jnp.float32),
    )(z2[0, :N], z2[1, :N], y2, dinv, b2[None, :], mask_f, Wr, br[None, :])

    return pred


# speedup vs baseline: 1.0202x; 1.0202x over previous
import functools

import jax
import jax.numpy as jnp
from jax import lax
from jax.experimental import pallas as pl
from jax.experimental.pallas import tpu as pltpu
from jax.experimental.pallas import tpu_sc as plsc

N = 10000
E = 320000
D = 128
DOUT = 12

NC = 2
NS = 16
NW = NC * NS
K = 128
NBUF = 2
CHUNKS = ((-(-E // (NW * K)) + NBUF - 1) // NBUF) * NBUF
EPAD = CHUNKS * NW * K
NPAD = (N // 128 + 1) * 128
RPT = NPAD // NS

_mesh = plsc.VectorSubcoreMesh(core_axis_name="c", subcore_axis_name="s")

def _fill_2d(ref, rows, val):
    val16 = jnp.full((16,), val, jnp.float32)
    def body(i, _):
        r = i // (D // 16)
        col = (i % (D // 16)) * 16
        ref[r, pl.ds(col, 16)] = val16
        return 0
    lax.fori_loop(0, rows * (D // 16), body, 0)


def _fill_1d(ref, n, val):
    val16 = jnp.full((16,), val, jnp.float32)
    def body(i, _):
        ref[pl.ds(i * 16, 16)] = val16
        return 0
    lax.fori_loop(0, n // 16, body, 0)


@functools.partial(
    pl.kernel,
    out_type=jax.ShapeDtypeStruct((NC * NPAD,), jnp.float32),
    mesh=_mesh,
    scratch_types=[
        pltpu.VMEM((CHUNKS, K), jnp.int32),
        pltpu.VMEM((K,), jnp.float32),
        pltpu.VMEM((((RPT + 15) // 16) * 16,), jnp.float32),
        pltpu.VMEM_SHARED((NPAD,), jnp.float32),
        pltpu.SemaphoreType.DMA,
        pltpu.SemaphoreType.DMA,
    ],
)
def _sc_degree(dst_hbm, out_hbm, didx_v, ones_v, zrow_v, acc, semi, sems):
    c = lax.axis_index("c")
    s = lax.axis_index("s")
    wid = s * NC + c

    pltpu.async_copy(dst_hbm.at[pl.ds(wid * CHUNKS, CHUNKS)], didx_v, semi)
    _fill_1d(ones_v, K, 1.0)
    _fill_1d(zrow_v, ((RPT + 15) // 16) * 16, 0.0)
    pltpu.sync_copy(zrow_v.at[pl.ds(0, RPT)], acc.at[pl.ds(s * RPT, RPT)])
    pltpu.make_async_copy(dst_hbm.at[pl.ds(0, CHUNKS)], didx_v, semi).wait()
    plsc.subcore_barrier()

    def body(j, _):
        pltpu.async_copy(ones_v, acc.at[didx_v.at[j]], sems, add=True)
        return 0

    lax.fori_loop(0, CHUNKS, body, 0)

    def drain(j, _):
        pltpu.make_async_copy(ones_v, acc.at[didx_v.at[0]], sems).wait()
        return 0

    lax.fori_loop(0, CHUNKS, drain, 0)
    plsc.subcore_barrier()
    pltpu.sync_copy(acc.at[pl.ds(s * RPT, RPT)], zrow_v.at[pl.ds(0, RPT)])
    pltpu.sync_copy(zrow_v.at[pl.ds(0, RPT)],
                    out_hbm.at[pl.ds(c * NPAD + s * RPT, RPT)])


@functools.partial(
    pl.kernel,
    out_type=jax.ShapeDtypeStruct((NC, NPAD, D), jnp.float32),
    mesh=_mesh,
    scratch_types=[
        pltpu.VMEM((2, NBUF, K), jnp.int32),
        pltpu.VMEM((2, NBUF, K), jnp.int32),
        pltpu.VMEM((NBUF, K, D), jnp.float32),
        pltpu.VMEM_SHARED((NPAD, D), jnp.float32),
        pltpu.SemaphoreType.DMA,
        [pltpu.SemaphoreType.DMA] * NBUF,
    ],
)
def _sc_segsum(y_hbm, src_hbm, dst_hbm, out_hbm, sidx_v, didx_v, rows_v, acc,
               semi, semg):
    c = lax.axis_index("c")
    s = lax.axis_index("s")
    wid = s * NC + c
    base = wid * CHUNKS
    G = CHUNKS // NBUF

    def istart(g, p):
        pltpu.async_copy(src_hbm.at[pl.ds(base + g * NBUF, NBUF)],
                         sidx_v.at[p], semi)
        pltpu.async_copy(dst_hbm.at[pl.ds(base + g * NBUF, NBUF)],
                         didx_v.at[p], semi)

    def iwait():
        pltpu.make_async_copy(src_hbm.at[pl.ds(0, NBUF)], sidx_v.at[0],
                              semi).wait()
        pltpu.make_async_copy(dst_hbm.at[pl.ds(0, NBUF)], didx_v.at[0],
                              semi).wait()

    istart(0, 0)

    val16 = jnp.zeros((16,), jnp.float32)

    def zbody(i, _):
        r = i // (D // 16)
        col = (i % (D // 16)) * 16
        rows_v[0, r, pl.ds(col, 16)] = val16
        return 0

    lax.fori_loop(0, K * (D // 16), zbody, 0)
    rbase = s * RPT
    for j in range(RPT // K):
        pltpu.sync_copy(rows_v.at[0], acc.at[pl.ds(rbase + j * K, K)])
    rem = RPT % K
    if rem:
        pltpu.sync_copy(rows_v.at[0, pl.ds(0, rem)],
                        acc.at[pl.ds(rbase + (RPT // K) * K, rem)])
    iwait()
    plsc.subcore_barrier()

    def gstart(p, b):
        pltpu.async_copy(y_hbm.at[sidx_v.at[p, b]], rows_v.at[b], semg[b])

    def gwait(b):
        pltpu.make_async_copy(y_hbm.at[sidx_v.at[0, 0]], rows_v.at[b],
                              semg[b]).wait()

    for b in range(NBUF):
        gstart(0, b)

    def outer(g, _):
        p = lax.rem(g, 2)
        pn = 1 - p
        more = g + 1 < G

        @pl.when(more)
        def _():
            istart(g + 1, pn)

        gwait(0)
        pltpu.sync_copy(rows_v.at[0], acc.at[didx_v.at[p, 0]], add=True)

        @pl.when(more)
        def _():
            iwait()
            gstart(pn, 0)

        gwait(1)
        pltpu.sync_copy(rows_v.at[1], acc.at[didx_v.at[p, 1]], add=True)

        @pl.when(more)
        def _():
            gstart(pn, 1)
        return 0

    lax.fori_loop(0, G, outer, 0)
    plsc.subcore_barrier()
    pltpu.sync_copy(acc.at[pl.ds(rbase, RPT)], out_hbm.at[c, pl.ds(rbase, RPT)])


_R = 1000


def _tc_dense1_body(x_ref, w_ref, c0_ref, c1_ref, y_ref, dinv_ref):
    deg = c0_ref[...] + c1_ref[...] + 1.0
    dinv = lax.rsqrt(deg)
    dinv_ref[...] = dinv
    y_ref[...] = jnp.dot(x_ref[...], w_ref[...],
                         preferred_element_type=jnp.float32) * dinv


def _tc_dense_mid_body(z0_ref, z1_ref, y_ref, dinv_ref, b_ref, w_ref, out_ref):
    dinv = dinv_ref[...]
    h = dinv * (z0_ref[...] + z1_ref[...] + y_ref[...]) + b_ref[...]
    h = jnp.maximum(h, 0.0)
    out_ref[...] = jnp.dot(h, w_ref[...],
                           preferred_element_type=jnp.float32) * dinv


def _tc_dense_out_body(z0_ref, z1_ref, y_ref, dinv_ref, b_ref, m_ref,
                       wr_ref, br_ref, out_ref):
    h = dinv_ref[...] * (z0_ref[...] + z1_ref[...] + y_ref[...]) + b_ref[...]
    h = jnp.maximum(h, 0.0) * m_ref[...]
    out_ref[...] = jnp.dot(h, wr_ref[...],
                           preferred_element_type=jnp.float32) + br_ref[...]


def _rows_spec(width):
    return pl.BlockSpec((_R, width), lambda i: (i, 0))


def _full_spec(shape):
    return pl.BlockSpec(shape, lambda i: (0,) * len(shape))


def kernel(x, edge_index, target_mask, W1, b1, W2, b2, Wr, br):
    src = edge_index[0]
    dst = edge_index[1]
    pad = EPAD - E
    src_p = jnp.concatenate([src, jnp.zeros((pad,), jnp.int32)])
    src_p = src_p.reshape(NW * CHUNKS, K)
    dst_p = jnp.concatenate([dst, jnp.full((pad,), N, jnp.int32)])
    dst_p = dst_p.reshape(NW * CHUNKS, K)

    cnt = _sc_degree(dst_p)
    c0 = cnt[:N, None]
    c1 = cnt[NPAD:NPAD + N, None]

    grid = N // _R
    y1, dinv = pl.pallas_call(
        _tc_dense1_body,
        grid=(grid,),
        in_specs=[_rows_spec(D), _full_spec((D, D)), _rows_spec(1), _rows_spec(1)],
        out_specs=[_rows_spec(D), _rows_spec(1)],
        out_shape=[jax.ShapeDtypeStruct((N, D), jnp.float32),
                   jax.ShapeDtypeStruct((N, 1), jnp.float32)],
    )(x, W1, c0, c1)

    z = _sc_segsum(y1, src_p, dst_p)

    y2 = pl.pallas_call(
        _tc_dense_mid_body,
        grid=(grid,),
        in_specs=[_rows_spec(D), _rows_spec(D), _rows_spec(D), _rows_spec(1),
                  _full_spec((1, D)), _full_spec((D, D))],
        out_specs=_rows_spec(D),
        out_shape=jax.ShapeDtypeStruct((N, D), jnp.float32),
    )(z[0, :N], z[1, :N], y1, dinv, b1[None, :], W2)

    z2 = _sc_segsum(y2, src_p, dst_p)

    mask_f = target_mask.astype(jnp.float32)[:, None]
    pred = pl.pallas_call(
        _tc_dense_out_body,
        grid=(grid,),
        in_specs=[_rows_spec(D), _rows_spec(D), _rows_spec(D), _rows_spec(1),
                  _full_spec((1, D)), _rows_spec(1),
                  _full_spec((D, DOUT)), _full_spec((1, DOUT))],
        out_specs=_rows_spec(DOUT),
        out_shape=jax.ShapeDtypeStruct((N, DOUT), jnp.float32),
    )(z2[0, :N], z2[1, :N], y2, dinv, b2[None, :], mask_f, Wr, br[None, :])

    return pred


# speedup vs baseline: 1.3128x; 1.2867x over previous
import functools

import jax
import jax.numpy as jnp
from jax import lax
from jax.experimental import pallas as pl
from jax.experimental.pallas import tpu as pltpu
from jax.experimental.pallas import tpu_sc as plsc

N = 10000
E = 320000
D = 128
DOUT = 12

NC = 2
NS = 16
NW = NC * NS
K = 128
NBUF = 2
CHUNKS = ((-(-E // (NW * K)) + NBUF - 1) // NBUF) * NBUF
EPAD = CHUNKS * NW * K
NPAD = (N // 128 + 1) * 128
RPT = NPAD // NS

_mesh = plsc.VectorSubcoreMesh(core_axis_name="c", subcore_axis_name="s")

def _fill_2d(ref, rows, val):
    val16 = jnp.full((16,), val, jnp.float32)
    def body(i, _):
        r = i // (D // 16)
        col = (i % (D // 16)) * 16
        ref[r, pl.ds(col, 16)] = val16
        return 0
    lax.fori_loop(0, rows * (D // 16), body, 0)


def _fill_1d(ref, n, val):
    val16 = jnp.full((16,), val, jnp.float32)
    def body(i, _):
        ref[pl.ds(i * 16, 16)] = val16
        return 0
    lax.fori_loop(0, n // 16, body, 0)


@functools.partial(
    pl.kernel,
    out_type=jax.ShapeDtypeStruct((NC * NPAD,), jnp.float32),
    mesh=_mesh,
    scratch_types=[
        pltpu.VMEM((CHUNKS, K), jnp.int32),
        pltpu.VMEM((K,), jnp.float32),
        pltpu.VMEM((((RPT + 15) // 16) * 16,), jnp.float32),
        pltpu.VMEM_SHARED((NPAD,), jnp.float32),
        pltpu.SemaphoreType.DMA,
        pltpu.SemaphoreType.DMA,
    ],
)
def _sc_degree(dst_hbm, out_hbm, didx_v, ones_v, zrow_v, acc, semi, sems):
    c = lax.axis_index("c")
    s = lax.axis_index("s")
    wid = s * NC + c

    pltpu.async_copy(dst_hbm.at[pl.ds(wid * CHUNKS, CHUNKS)], didx_v, semi)
    _fill_1d(ones_v, K, 1.0)
    _fill_1d(zrow_v, ((RPT + 15) // 16) * 16, 0.0)
    pltpu.sync_copy(zrow_v.at[pl.ds(0, RPT)], acc.at[pl.ds(s * RPT, RPT)])
    pltpu.make_async_copy(dst_hbm.at[pl.ds(0, CHUNKS)], didx_v, semi).wait()
    plsc.subcore_barrier()

    def body(j, _):
        pltpu.async_copy(ones_v, acc.at[didx_v.at[j]], sems, add=True)
        return 0

    lax.fori_loop(0, CHUNKS, body, 0)

    def drain(j, _):
        pltpu.make_async_copy(ones_v, acc.at[didx_v.at[0]], sems).wait()
        return 0

    lax.fori_loop(0, CHUNKS, drain, 0)
    plsc.subcore_barrier()
    pltpu.sync_copy(acc.at[pl.ds(s * RPT, RPT)], zrow_v.at[pl.ds(0, RPT)])
    pltpu.sync_copy(zrow_v.at[pl.ds(0, RPT)],
                    out_hbm.at[pl.ds(c * NPAD + s * RPT, RPT)])


@functools.partial(
    pl.kernel,
    out_type=jax.ShapeDtypeStruct((NC, NPAD, D), jnp.float32),
    mesh=_mesh,
    scratch_types=[
        pltpu.VMEM((2, NBUF, K), jnp.int32),
        pltpu.VMEM((2, NBUF, K), jnp.int32),
        pltpu.VMEM((NBUF, K, D), jnp.float32),
        pltpu.VMEM_SHARED((NPAD, D), jnp.float32),
        pltpu.SemaphoreType.DMA,
        [pltpu.SemaphoreType.DMA] * NBUF,
    ],
)
def _sc_segsum(y_hbm, src_hbm, dst_hbm, out_hbm, sidx_v, didx_v, rows_v, acc,
               semi, semg):
    c = lax.axis_index("c")
    s = lax.axis_index("s")
    wid = s * NC + c
    base = wid * CHUNKS
    G = CHUNKS // NBUF

    def istart(g, p):
        pltpu.async_copy(src_hbm.at[pl.ds(base + g * NBUF, NBUF)],
                         sidx_v.at[p], semi)
        pltpu.async_copy(dst_hbm.at[pl.ds(base + g * NBUF, NBUF)],
                         didx_v.at[p], semi)

    def iwait():
        pltpu.make_async_copy(src_hbm.at[pl.ds(0, NBUF)], sidx_v.at[0],
                              semi).wait()
        pltpu.make_async_copy(dst_hbm.at[pl.ds(0, NBUF)], didx_v.at[0],
                              semi).wait()

    istart(0, 0)

    val16 = jnp.zeros((16,), jnp.float32)

    def zbody(i, _):
        r = i // (D // 16)
        col = (i % (D // 16)) * 16
        rows_v[0, r, pl.ds(col, 16)] = val16
        return 0

    lax.fori_loop(0, K * (D // 16), zbody, 0)
    rbase = s * RPT
    for j in range(RPT // K):
        pltpu.sync_copy(rows_v.at[0], acc.at[pl.ds(rbase + j * K, K)])
    rem = RPT % K
    if rem:
        pltpu.sync_copy(rows_v.at[0, pl.ds(0, rem)],
                        acc.at[pl.ds(rbase + (RPT // K) * K, rem)])
    iwait()
    plsc.subcore_barrier()

    def gstart(p, b):
        pltpu.async_copy(y_hbm.at[sidx_v.at[p, b]], rows_v.at[b], semg[b])

    def gwait(b):
        pltpu.make_async_copy(y_hbm.at[sidx_v.at[0, 0]], rows_v.at[b],
                              semg[b]).wait()

    for b in range(NBUF):
        gstart(0, b)

    def outer(g, _):
        p = lax.rem(g, 2)
        pn = 1 - p
        more = g + 1 < G

        @pl.when(more)
        def _():
            istart(g + 1, pn)

        gwait(0)
        pltpu.sync_copy(rows_v.at[0], acc.at[didx_v.at[p, 0]], add=True)

        @pl.when(more)
        def _():
            iwait()
            gstart(pn, 0)

        gwait(1)
        pltpu.sync_copy(rows_v.at[1], acc.at[didx_v.at[p, 1]], add=True)

        @pl.when(more)
        def _():
            gstart(pn, 1)
        return 0

    lax.fori_loop(0, G, outer, 0)
    plsc.subcore_barrier()
    pltpu.sync_copy(acc.at[pl.ds(rbase, RPT)], out_hbm.at[c, pl.ds(rbase, RPT)])


_R = 1000


def _tc_dense1_body(x_ref, w_ref, c0_ref, c1_ref, y_ref, dinv_ref):
    deg = c0_ref[...] + c1_ref[...] + 1.0
    dinv = lax.rsqrt(deg)
    dinv_ref[...] = dinv
    y_ref[...] = jnp.dot(x_ref[...], w_ref[...],
                         preferred_element_type=jnp.float32) * dinv


def _tc_dense_mid_body(z0_ref, z1_ref, y_ref, dinv_ref, b_ref, w_ref, out_ref):
    dinv = dinv_ref[...]
    h = dinv * (z0_ref[...] + z1_ref[...] + y_ref[...]) + b_ref[...]
    h = jnp.maximum(h, 0.0)
    out_ref[...] = jnp.dot(h, w_ref[...],
                           preferred_element_type=jnp.float32) * dinv


def _tc_dense_out_body(z0_ref, z1_ref, y_ref, dinv_ref, b_ref, m_ref,
                       wr_ref, br_ref, out_ref):
    h = dinv_ref[...] * (z0_ref[...] + z1_ref[...] + y_ref[...]) + b_ref[...]
    h = jnp.maximum(h, 0.0) * m_ref[...]
    out_ref[...] = jnp.dot(h, wr_ref[...],
                           preferred_element_type=jnp.float32) + br_ref[...]


def _rows_spec(width):
    return pl.BlockSpec((_R, width), lambda i: (i, 0))


def _full_spec(shape):
    return pl.BlockSpec(shape, lambda i: (0,) * len(shape))


def kernel(x, edge_index, target_mask, W1, b1, W2, b2, Wr, br):
    ept = E // NW
    ppt = CHUNKS * K - ept
    pad_src = jnp.zeros((NW, ppt), jnp.int32)
    pad_dst = (jnp.arange(NW * ppt, dtype=jnp.int32) % (NPAD - N) + N
               ).reshape(NW, ppt)
    src_p = jnp.concatenate([edge_index[0].reshape(NW, ept), pad_src], axis=1)
    src_p = src_p.reshape(NW * CHUNKS, K)
    dst_p = jnp.concatenate([edge_index[1].reshape(NW, ept), pad_dst], axis=1)
    dst_p = dst_p.reshape(NW * CHUNKS, K)

    cnt = _sc_degree(dst_p)
    c0 = cnt[:N, None]
    c1 = cnt[NPAD:NPAD + N, None]

    grid = N // _R
    y1, dinv = pl.pallas_call(
        _tc_dense1_body,
        grid=(grid,),
        in_specs=[_rows_spec(D), _full_spec((D, D)), _rows_spec(1), _rows_spec(1)],
        out_specs=[_rows_spec(D), _rows_spec(1)],
        out_shape=[jax.ShapeDtypeStruct((N, D), jnp.float32),
                   jax.ShapeDtypeStruct((N, 1), jnp.float32)],
    )(x, W1, c0, c1)

    z = _sc_segsum(y1, src_p, dst_p)

    y2 = pl.pallas_call(
        _tc_dense_mid_body,
        grid=(grid,),
        in_specs=[_rows_spec(D), _rows_spec(D), _rows_spec(D), _rows_spec(1),
                  _full_spec((1, D)), _full_spec((D, D))],
        out_specs=_rows_spec(D),
        out_shape=jax.ShapeDtypeStruct((N, D), jnp.float32),
    )(z[0, :N], z[1, :N], y1, dinv, b1[None, :], W2)

    z2 = _sc_segsum(y2, src_p, dst_p)

    mask_f = target_mask.astype(jnp.float32)[:, None]
    pred = pl.pallas_call(
        _tc_dense_out_body,
        grid=(grid,),
        in_specs=[_rows_spec(D), _rows_spec(D), _rows_spec(D), _rows_spec(1),
                  _full_spec((1, D)), _rows_spec(1),
                  _full_spec((D, DOUT)), _full_spec((1, DOUT))],
        out_specs=_rows_spec(DOUT),
        out_shape=jax.ShapeDtypeStruct((N, DOUT), jnp.float32),
    )(z2[0, :N], z2[1, :N], y2, dinv, b2[None, :], mask_f, Wr, br[None, :])

    return pred
